# baseline (speedup 1.0000x reference)
import functools

import jax
import jax.numpy as jnp
from jax import lax
from jax.experimental import pallas as pl
from jax.experimental.pallas import tpu as pltpu
from jax.experimental.pallas import tpu_sc as plsc

N_EMB = 8192
DIM = 32
ROWS = 8192
TILE = 512
GRID = ROWS // TILE
COMMIT = 0.25
KLD = 100.0

_NC = 2
_NS = 16
_NW = _NC * _NS
_B_PER_W = ROWS // _NW


_GATHER_W = 128
_CHUNK = 128


def _sc_gather_kernel(cb_hbm, idx_hbm, out_hbm, idx_v, rows_v, sem):
    wid = lax.axis_index("s") * _NC + lax.axis_index("c")
    base = wid * _B_PER_W
    pltpu.sync_copy(idx_hbm.at[pl.ds(base, _B_PER_W)], idx_v)
    for c in range(_B_PER_W // _CHUNK):
        pltpu.async_copy(
            cb_hbm.at[idx_v.at[pl.ds(c * _CHUNK, _CHUNK)]],
            rows_v.at[pl.ds(c * _CHUNK, _CHUNK)], sem).wait()
    pltpu.sync_copy(rows_v, out_hbm.at[pl.ds(base, _B_PER_W)])


def _sc_gather(codebook_padded, idx):
    mesh = plsc.VectorSubcoreMesh(core_axis_name="c", subcore_axis_name="s")
    return pl.kernel(
        _sc_gather_kernel,
        mesh=mesh,
        out_type=jax.ShapeDtypeStruct((ROWS, _GATHER_W), jnp.float32),
        scratch_types=[
            pltpu.VMEM((_B_PER_W,), jnp.int32),
            pltpu.VMEM((_B_PER_W, _GATHER_W), jnp.float32),
            pltpu.SemaphoreType.DMA,
        ],
    )(codebook_padded, idx)


def _vq_kernel(x_ref, idx_ref, quant_ref,
               enc_ref, loss_ref, perp_ref,
               counts_ref, sq_ref):
    i = pl.program_id(0)

    @pl.when(i == 0)
    def _init():
        counts_ref[...] = jnp.zeros_like(counts_ref)
        sq_ref[0, 0] = 0.0

    x = x_ref[...]
    idx = idx_ref[...]
    onehot = (jax.lax.broadcasted_iota(jnp.int32, (TILE, N_EMB), 1)
              == idx).astype(jnp.float32)
    enc_ref[...] = onehot
    counts_ref[...] += jnp.sum(onehot, axis=0, keepdims=True)
    d = quant_ref[...] - x
    sq_ref[0, 0] += jnp.sum(d * d)

    @pl.when(i == GRID - 1)
    def _finalize():
        mse = sq_ref[0, 0] / float(ROWS * DIM)
        loss_ref[0, 0] = (1.0 + COMMIT) * mse * KLD
        avg = counts_ref[...] / float(ROWS)
        ent = jnp.sum(avg * jnp.log(avg + 1e-10))
        perp_ref[0, 0] = jnp.exp(-ent)


@functools.partial(jax.jit, static_argnames=())
def kernel(inputs, codebook):
    x = jnp.transpose(inputs, (0, 2, 1))
    flat = x.reshape(-1, DIM)
    distances = (jnp.sum(flat ** 2, axis=1, keepdims=True)
                 + jnp.sum(codebook ** 2, axis=1)
                 - 2.0 * jnp.matmul(flat, codebook.T))
    encoding_indices = jnp.argmin(distances, axis=1)
    idx32 = encoding_indices.astype(jnp.int32)

    cb_padded = jnp.pad(codebook, ((0, 0), (0, _GATHER_W - DIM)))
    quant = _sc_gather(cb_padded, idx32)[:, :DIM]

    enc, loss, perp = pl.pallas_call(
        _vq_kernel,
        grid=(GRID,),
        in_specs=[
            pl.BlockSpec((TILE, DIM), lambda i: (i, 0)),
            pl.BlockSpec((TILE, 1), lambda i: (i, 0)),
            pl.BlockSpec((TILE, DIM), lambda i: (i, 0)),
        ],
        out_specs=[
            pl.BlockSpec((TILE, N_EMB), lambda i: (i, 0)),
            pl.BlockSpec((1, 1), lambda i: (0, 0), memory_space=pltpu.SMEM),
            pl.BlockSpec((1, 1), lambda i: (0, 0), memory_space=pltpu.SMEM),
        ],
        out_shape=[
            jax.ShapeDtypeStruct((ROWS, N_EMB), jnp.float32),
            jax.ShapeDtypeStruct((1, 1), jnp.float32),
            jax.ShapeDtypeStruct((1, 1), jnp.float32),
        ],
        scratch_shapes=[
            pltpu.VMEM((1, N_EMB), jnp.float32),
            pltpu.SMEM((1, 1), jnp.float32),
        ],
        compiler_params=pltpu.CompilerParams(
            dimension_semantics=("arbitrary",),
        ),
    )(flat, idx32[:, None], quant)

    quant_out = jnp.transpose(quant.reshape(8, 1024, DIM), (0, 2, 1))
    return (loss[0, 0], quant_out, perp[0, 0], enc)


# speedup vs baseline: 1.0044x; 1.0044x over previous
import functools

import jax
import jax.numpy as jnp
from jax import lax
from jax.experimental import pallas as pl
from jax.experimental.pallas import tpu as pltpu
from jax.experimental.pallas import tpu_sc as plsc

N_EMB = 8192
DIM = 32
ROWS = 8192
TILE = 512
GRID = ROWS // TILE
COMMIT = 0.25
KLD = 100.0

_NC = 2
_NS = 16
_NW = _NC * _NS
_B_PER_W = ROWS // _NW
_GATHER_W = 128
_CHUNK = 128
_LANES = 16


def _sc_kernel(cb_hbm, idx_hbm, x_hbm, quant_hbm, sq_hbm,
               idx_v, rows_v, x_v, acc_v, sem):
    wid = lax.axis_index("s") * _NC + lax.axis_index("c")
    base = wid * _B_PER_W
    pltpu.sync_copy(idx_hbm.at[pl.ds(base, _B_PER_W)], idx_v)
    for c in range(_B_PER_W // _CHUNK):
        pltpu.async_copy(
            cb_hbm.at[idx_v.at[pl.ds(c * _CHUNK, _CHUNK)]],
            rows_v.at[pl.ds(c * _CHUNK, _CHUNK)], sem).wait()
    pltpu.sync_copy(x_hbm.at[pl.ds(base, _B_PER_W)], x_v)
    pltpu.sync_copy(rows_v, quant_hbm.at[pl.ds(base, _B_PER_W)])

    def body(i, acc):
        for c in range(DIM // _LANES):
            q = rows_v[i, pl.ds(c * _LANES, _LANES)]
            xx = x_v[i, pl.ds(c * _LANES, _LANES)]
            d = q - xx
            acc = acc + d * d
        return acc

    acc = lax.fori_loop(0, _B_PER_W, body, jnp.zeros((_LANES,), jnp.float32))
    acc_v[...] = acc
    pltpu.sync_copy(acc_v, sq_hbm.at[wid])


def _sc_gather_sq(codebook_padded, idx, flat):
    mesh = plsc.VectorSubcoreMesh(core_axis_name="c", subcore_axis_name="s")
    return pl.kernel(
        _sc_kernel,
        mesh=mesh,
        out_type=[
            jax.ShapeDtypeStruct((ROWS, _GATHER_W), jnp.float32),
            jax.ShapeDtypeStruct((_NW, _LANES), jnp.float32),
        ],
        scratch_types=[
            pltpu.VMEM((_B_PER_W,), jnp.int32),
            pltpu.VMEM((_B_PER_W, _GATHER_W), jnp.float32),
            pltpu.VMEM((_B_PER_W, DIM), jnp.float32),
            pltpu.VMEM((_LANES,), jnp.float32),
            pltpu.SemaphoreType.DMA,
        ],
    )(codebook_padded, idx, flat)


def _vq_kernel(idx_ref, enc_ref, perp_ref, counts_ref):
    i = pl.program_id(0)

    @pl.when(i == 0)
    def _init():
        counts_ref[...] = jnp.zeros_like(counts_ref)

    idx = idx_ref[...]
    onehot = (jax.lax.broadcasted_iota(jnp.int32, (TILE, N_EMB), 1)
              == idx).astype(jnp.float32)
    enc_ref[...] = onehot
    counts_ref[...] += jnp.sum(onehot, axis=0, keepdims=True)

    @pl.when(i == GRID - 1)
    def _finalize():
        avg = counts_ref[...] / float(ROWS)
        ent = jnp.sum(avg * jnp.log(avg + 1e-10))
        perp_ref[0, 0] = jnp.exp(-ent)


@functools.partial(jax.jit, static_argnames=())
def kernel(inputs, codebook):
    x = jnp.transpose(inputs, (0, 2, 1))
    flat = x.reshape(-1, DIM)
    distances = (jnp.sum(flat ** 2, axis=1, keepdims=True)
                 + jnp.sum(codebook ** 2, axis=1)
                 - 2.0 * jnp.matmul(flat, codebook.T))
    encoding_indices = jnp.argmin(distances, axis=1)
    idx32 = encoding_indices.astype(jnp.int32)

    cb_padded = jnp.pad(codebook, ((0, 0), (0, _GATHER_W - DIM)))
    quant_pad, sq_parts = _sc_gather_sq(cb_padded, idx32, flat)
    quant = quant_pad[:, :DIM]

    enc, perp = pl.pallas_call(
        _vq_kernel,
        grid=(GRID,),
        in_specs=[
            pl.BlockSpec((TILE, 1), lambda i: (i, 0)),
        ],
        out_specs=[
            pl.BlockSpec((TILE, N_EMB), lambda i: (i, 0)),
            pl.BlockSpec((1, 1), lambda i: (0, 0), memory_space=pltpu.SMEM),
        ],
        out_shape=[
            jax.ShapeDtypeStruct((ROWS, N_EMB), jnp.float32),
            jax.ShapeDtypeStruct((1, 1), jnp.float32),
        ],
        scratch_shapes=[
            pltpu.VMEM((1, N_EMB), jnp.float32),
        ],
        compiler_params=pltpu.CompilerParams(
            dimension_semantics=("arbitrary",),
        ),
    )(idx32[:, None])

    mse = jnp.sum(sq_parts) / float(ROWS * DIM)
    loss = (1.0 + COMMIT) * mse * KLD
    quant_out = jnp.transpose(quant.reshape(8, 1024, DIM), (0, 2, 1))
    return (loss, quant_out, perp[0, 0], enc)
